# baseline (speedup 1.0000x reference)
import jax
import jax.numpy as jnp
import numpy as np
from jax.experimental import pallas as pl

N = 50000
E = 800000
D = 31
HID = 8
NLAYERS = 14
LIG = 139
L = 140
DH = 62
NH = 2


def _twirls_prop(h_in, src, dst, norm):
    Y = h_in
    for _ in range(7):
        diff = Y[src] - Y[dst]
        w = 1.0 / jnp.sqrt(jnp.sum(diff * diff, axis=-1) + 1.0)
        msg = (w * norm)[:, None] * Y[src]
        agg = jax.ops.segment_sum(msg, dst, num_segments=N)
        Y = (h_in + agg) * 0.5
    return Y


def _lstm_dir(xs, Wih, Whh, bih, bhh, reverse):
    def step(carry, xt):
        h, c = carry
        g = xt @ Wih.T + h @ Whh.T + bih + bhh
        i, f, gg, o = jnp.split(g, 4)
        i = jax.nn.sigmoid(i)
        f = jax.nn.sigmoid(f)
        gg = jnp.tanh(gg)
        o = jax.nn.sigmoid(o)
        c = f * c + i * gg
        h = o * jnp.tanh(c)
        return (h, c), h
    seq = xs[::-1] if reverse else xs
    init = (jnp.zeros((D,), xs.dtype), jnp.zeros((D,), xs.dtype))
    _, hs = jax.lax.scan(step, init, seq)
    return hs[::-1] if reverse else hs


def _identity_pallas(x):
    def body(x_ref, o_ref):
        o_ref[...] = x_ref[...]
    return pl.pallas_call(
        body, out_shape=jax.ShapeDtypeStruct(x.shape, x.dtype))(x)


def kernel(x, edge_index, ligand_emb, params):
    src = edge_index[0]
    dst = edge_index[1]
    ones_e = jnp.ones((E,), jnp.float32)
    deg_out = jax.ops.segment_sum(ones_e, src, num_segments=N)
    deg_in = jax.ops.segment_sum(ones_e, dst, num_segments=N)
    norm = 1.0 / jnp.sqrt((deg_out[src] + 1.0) * (deg_in[dst] + 1.0))
    feat = x
    for i in range(NLAYERS):
        h = jax.nn.relu(feat @ params['convW1'][i] + params['convb1'][i])
        h = h @ params['convW2'][i] + params['convb2'][i]
        feat = jax.nn.relu(_twirls_prop(h, src, dst, norm))
    gate = feat @ params['pool_w'] + params['pool_b']
    a = jax.nn.softmax(gate, axis=0)
    protein_rep = jnp.sum(a * feat, axis=0, keepdims=True)
    seq = jnp.concatenate([ligand_emb, protein_rep], axis=0)
    mask = jnp.eye(L, dtype=jnp.float32)
    mask = mask.at[:, L - 1].set(1.0)
    mask = mask.at[L - 1, :].set(1.0)
    mask = mask.at[L - 1, L - 1].set(0.0)
    h = seq
    for layer in (0, 1):
        sf = '_l' + str(layer) + 'f'
        sb = '_l' + str(layer) + 'b'
        hf = _lstm_dir(h, params['Wih' + sf], params['Whh' + sf], params['bih' + sf], params['bhh' + sf], False)
        hb = _lstm_dir(h, params['Wih' + sb], params['Whh' + sb], params['bih' + sb], params['bhh' + sb], True)
        h = jnp.concatenate([hf, hb], axis=-1)
    out = h[None]
    dh = DH // NH
    q = (out @ params['Wq'] + params['bq']).reshape(1, L, NH, dh).transpose(0, 2, 1, 3)
    k = (out @ params['Wk'] + params['bk']).reshape(1, L, NH, dh).transpose(0, 2, 1, 3)
    v = (out @ params['Wv'] + params['bv']).reshape(1, L, NH, dh).transpose(0, 2, 1, 3)
    scores = (q @ k.transpose(0, 1, 3, 2)) / float(np.sqrt(dh))
    scores = jnp.where(mask[None, None, :, :] > 0, scores, -1e9)
    attn = jax.nn.softmax(scores, axis=-1)
    ctx = (attn @ v).transpose(0, 2, 1, 3).reshape(1, L, DH)
    ctx = ctx @ params['Wo'] + params['bo']
    flat = ctx.reshape(1, L * DH)
    hid = jax.nn.relu(flat @ params['fc_in_w'] + params['fc_in_b'])
    return _identity_pallas(jax.nn.sigmoid(hid @ params['fc_out_w'] + params['fc_out_b']))


# speedup vs baseline: 2.5808x; 2.5808x over previous
import functools

import jax
import jax.numpy as jnp
import numpy as np
from jax import lax
from jax.experimental import pallas as pl
from jax.experimental.pallas import tpu as pltpu
from jax.experimental.pallas import tpu_sc as plsc

N = 50000
E = 800000
D = 31
HID = 8
NLAYERS = 14
LIG = 139
L = 140
DH = 62
NH = 2

NTILES = 32
NT = 1568
N_PAD = NTILES * NT
DP = 32
C = 512
GS = 128
E_PAD = E + 768
BU = 224


def _splat16(s):
    return jnp.full((16,), s, jnp.int32)


def _step_body(y_hbm, h_hbm, src_hbm, dst_hbm, wn_hbm, starts_hbm,
               ynew_hbm,
               sbuf, srcb, dstb, wnb, rowb, wsb, ysrc, ydst, agg,
               hbuf, ybuf, sem):
    wid = lax.axis_index("s") * 2 + lax.axis_index("c")
    base = wid * NT

    zero16 = jnp.zeros((16,), jnp.float32)

    def _zero(r, _):
        agg[r, pl.ds(0, 16)] = zero16
        agg[r, pl.ds(16, 16)] = zero16
        return 0
    lax.fori_loop(0, NT + 8, _zero, 0)

    pltpu.sync_copy(starts_hbm.at[wid], sbuf)
    start = sbuf[...][0]
    pltpu.sync_copy(starts_hbm.at[wid + 1], sbuf)
    end = sbuf[...][0]
    start_al = (start // 8) * 8
    nch = (end - start_al + (C - 1)) // C

    iota16 = lax.iota(jnp.int32, 16)

    def _chunk(c, _):
        off = start_al + c * C
        pltpu.sync_copy(src_hbm.at[pl.ds(off, C)], srcb)
        pltpu.sync_copy(dst_hbm.at[pl.ds(off, C)], dstb)
        pltpu.sync_copy(wn_hbm.at[pl.ds(off, C)], wnb)
        for j in range(C // GS):
            sl = pl.ds(j * GS, GS)
            pltpu.async_copy(y_hbm.at[srcb.at[sl]], ysrc.at[sl], sem).wait()
            pltpu.async_copy(y_hbm.at[dstb.at[sl]], ydst.at[sl], sem).wait()

        def _wgt(g, _):
            gb = g * 16
            dv = dstb[pl.ds(gb, 16)] - base
            ok = (dv >= 0) & (dv < NT)
            rowb[pl.ds(gb, 16)] = jnp.where(ok, dv, NT)
            ridx = gb + iota16
            acc = jnp.zeros((16,), jnp.float32)
            for d in range(DP):
                colv = _splat16(d)
                a = plsc.load_gather(ysrc, [ridx, colv])
                b = plsc.load_gather(ydst, [ridx, colv])
                df = a - b
                acc = acc + df * df
            x = acc + 1.0
            ib = plsc.bitcast(x, jnp.int32)
            yv = plsc.bitcast(jnp.int32(0x5F3759DF) - (ib >> 1), jnp.float32)
            for _ in range(3):
                yv = yv * (1.5 - 0.5 * x * yv * yv)
            wsb[pl.ds(gb, 16)] = yv * wnb[pl.ds(gb, 16)]
            return 0
        lax.fori_loop(0, C // 16, _wgt, 0)

        def _scat(e, _):
            rsp = plsc.load_gather(rowb, [_splat16(e)])
            wsp = plsc.load_gather(wsb, [_splat16(e)])
            lo = ysrc[e, pl.ds(0, 16)]
            hi = ysrc[e, pl.ds(16, 16)]
            plsc.addupdate_scatter(agg, [rsp, iota16], lo * wsp)
            plsc.addupdate_scatter(agg, [rsp, iota16 + 16], hi * wsp)
            return 0
        lax.fori_loop(0, C, _scat, 0)
        return 0

    lax.fori_loop(0, nch, _chunk, 0)

    def _upd(b, _):
        rb = base + b * BU
        pltpu.sync_copy(h_hbm.at[pl.ds(rb, BU)], hbuf)

        def _row(r, _):
            ar = b * BU + r
            ybuf[r, pl.ds(0, 16)] = (hbuf[r, pl.ds(0, 16)]
                                     + agg[ar, pl.ds(0, 16)]) * 0.5
            ybuf[r, pl.ds(16, 16)] = (hbuf[r, pl.ds(16, 16)]
                                      + agg[ar, pl.ds(16, 16)]) * 0.5
            return 0
        lax.fori_loop(0, BU, _row, 0)
        pltpu.sync_copy(ybuf, ynew_hbm.at[pl.ds(rb, BU)])
        return 0
    lax.fori_loop(0, NT // BU, _upd, 0)


@functools.partial(
    pl.kernel,
    out_type=jax.ShapeDtypeStruct((N_PAD, DP), jnp.float32),
    mesh=plsc.VectorSubcoreMesh(core_axis_name="c", subcore_axis_name="s"),
    scratch_types=[
        pltpu.VMEM((16,), jnp.int32),
        pltpu.VMEM((C,), jnp.int32),
        pltpu.VMEM((C,), jnp.int32),
        pltpu.VMEM((C,), jnp.float32),
        pltpu.VMEM((C,), jnp.int32),
        pltpu.VMEM((C,), jnp.float32),
        pltpu.VMEM((C, DP), jnp.float32),
        pltpu.VMEM((C, DP), jnp.float32),
        pltpu.VMEM((NT + 8, DP), jnp.float32),
        pltpu.VMEM((BU, DP), jnp.float32),
        pltpu.VMEM((BU, DP), jnp.float32),
        pltpu.SemaphoreType.DMA,
    ],
    compiler_params=pltpu.CompilerParams(needs_layout_passes=False,
                                         use_tc_tiling_on_sc=False),
)
def _prop_step(y_hbm, h_hbm, src_hbm, dst_hbm, wn_hbm, starts_hbm,
               ynew_hbm, *scratch):
    _step_body(y_hbm, h_hbm, src_hbm, dst_hbm, wn_hbm, starts_hbm,
               ynew_hbm, *scratch)


def _lstm_dir(xs, Wih, Whh, bih, bhh, reverse):
    def step(carry, xt):
        h, c = carry
        g = xt @ Wih.T + h @ Whh.T + bih + bhh
        i, f, gg, o = jnp.split(g, 4)
        i = jax.nn.sigmoid(i)
        f = jax.nn.sigmoid(f)
        gg = jnp.tanh(gg)
        o = jax.nn.sigmoid(o)
        c = f * c + i * gg
        h = o * jnp.tanh(c)
        return (h, c), h
    seq = xs[::-1] if reverse else xs
    init = (jnp.zeros((D,), xs.dtype), jnp.zeros((D,), xs.dtype))
    _, hs = jax.lax.scan(step, init, seq)
    return hs[::-1] if reverse else hs


def kernel(x, edge_index, ligand_emb, params):
    src = edge_index[0]
    dst = edge_index[1]

    order = jnp.argsort(dst)
    dst_s = dst[order]
    src_s = src[order]
    grid = jnp.arange(N + 1, dtype=jnp.int32)
    b_in = jnp.searchsorted(dst_s, grid)
    deg_in = jnp.diff(b_in)
    src_sorted = jnp.sort(src)
    b_out = jnp.searchsorted(src_sorted, grid)
    deg_out = jnp.diff(b_out)
    norm_s = jax.lax.rsqrt(
        (deg_out[src_s].astype(jnp.float32) + 1.0)
        * (deg_in[dst_s].astype(jnp.float32) + 1.0))

    tbound = jnp.searchsorted(dst_s, jnp.arange(33, dtype=jnp.int32) * NT)
    starts = jnp.full((40,), E, jnp.int32).at[:33].set(tbound.astype(jnp.int32))
    starts_b = jnp.broadcast_to(starts[:, None], (40, 16)).astype(jnp.int32)

    pad_e = E_PAD - E
    src_p = jnp.concatenate([src_s.astype(jnp.int32),
                             jnp.zeros((pad_e,), jnp.int32)])
    dst_p = jnp.concatenate([dst_s.astype(jnp.int32),
                             jnp.full((pad_e,), N_PAD - 1, jnp.int32)])
    wn_p = jnp.concatenate([norm_s, jnp.zeros((pad_e,), jnp.float32)])

    def pad_feat(f31):
        return jnp.pad(f31, ((0, N_PAD - N), (0, DP - D)))

    feat31 = x
    for i in range(NLAYERS):
        h31 = jax.nn.relu(feat31 @ params['convW1'][i] + params['convb1'][i])
        h31 = h31 @ params['convW2'][i] + params['convb2'][i]
        hp = pad_feat(h31)
        Y = hp
        for _ in range(7):
            Y = _prop_step(Y, hp, src_p, dst_p, wn_p, starts_b)
        feat31 = jax.nn.relu(Y[:N, :D])

    gate = feat31 @ params['pool_w'] + params['pool_b']
    a = jax.nn.softmax(gate, axis=0)
    protein_rep = jnp.sum(a * feat31, axis=0, keepdims=True)

    seq = jnp.concatenate([ligand_emb, protein_rep], axis=0)
    mask = jnp.eye(L, dtype=jnp.float32)
    mask = mask.at[:, L - 1].set(1.0)
    mask = mask.at[L - 1, :].set(1.0)
    mask = mask.at[L - 1, L - 1].set(0.0)
    h = seq
    for layer in (0, 1):
        sf = '_l' + str(layer) + 'f'
        sb = '_l' + str(layer) + 'b'
        hf = _lstm_dir(h, params['Wih' + sf], params['Whh' + sf],
                       params['bih' + sf], params['bhh' + sf], False)
        hb = _lstm_dir(h, params['Wih' + sb], params['Whh' + sb],
                       params['bih' + sb], params['bhh' + sb], True)
        h = jnp.concatenate([hf, hb], axis=-1)
    out = h[None]
    dh = DH // NH
    q = (out @ params['Wq'] + params['bq']).reshape(1, L, NH, dh).transpose(0, 2, 1, 3)
    k = (out @ params['Wk'] + params['bk']).reshape(1, L, NH, dh).transpose(0, 2, 1, 3)
    v = (out @ params['Wv'] + params['bv']).reshape(1, L, NH, dh).transpose(0, 2, 1, 3)
    scores = (q @ k.transpose(0, 1, 3, 2)) / float(np.sqrt(dh))
    scores = jnp.where(mask[None, None, :, :] > 0, scores, -1e9)
    attn = jax.nn.softmax(scores, axis=-1)
    ctx = (attn @ v).transpose(0, 2, 1, 3).reshape(1, L, DH)
    ctx = ctx @ params['Wo'] + params['bo']
    flat = ctx.reshape(1, L * DH)
    hid = jax.nn.relu(flat @ params['fc_in_w'] + params['fc_in_b'])
    return jax.nn.sigmoid(hid @ params['fc_out_w'] + params['fc_out_b'])


# speedup vs baseline: 3.5333x; 1.3691x over previous
import functools

import jax
import jax.numpy as jnp
import numpy as np
from jax import lax
from jax.experimental import pallas as pl
from jax.experimental.pallas import tpu as pltpu
from jax.experimental.pallas import tpu_sc as plsc

N = 50000
E = 800000
D = 31
HID = 8
NLAYERS = 14
LIG = 139
L = 140
DH = 62
NH = 2

NTILES = 32
NT = 1568
N_PAD = NTILES * NT
DP = 32
C = 512
GS = 128
E_PAD = E + 768
BU = 224


def _splat16(s):
    return jnp.full((16,), s, jnp.int32)


def _step_body(y_hbm, h_hbm, src_hbm, dst_hbm, wn_hbm, starts_hbm,
               ynew_hbm,
               sbuf, srcb, dstb, wnb, rowb, wsb, ysrc, ydst, agg,
               hbuf, ybuf, sem):
    wid = lax.axis_index("s") * 2 + lax.axis_index("c")
    base = wid * NT

    zero16 = jnp.zeros((16,), jnp.float32)

    @plsc.parallel_loop(0, NT + 8, unroll=4)
    def _zero(r):
        agg[r, pl.ds(0, 16)] = zero16
        agg[r, pl.ds(16, 16)] = zero16

    pltpu.sync_copy(starts_hbm.at[wid], sbuf)
    start = sbuf[...][0]
    pltpu.sync_copy(starts_hbm.at[wid + 1], sbuf)
    end = sbuf[...][0]
    start_al = (start // 8) * 8
    nch = (end - start_al + (C - 1)) // C

    iota16 = lax.iota(jnp.int32, 16)

    def _chunk(c, _):
        off = start_al + c * C
        e1 = pltpu.async_copy(src_hbm.at[pl.ds(off, C)], srcb, sem)
        e2 = pltpu.async_copy(dst_hbm.at[pl.ds(off, C)], dstb, sem)
        e3 = pltpu.async_copy(wn_hbm.at[pl.ds(off, C)], wnb, sem)
        e1.wait()
        e2.wait()
        e3.wait()
        descs = []
        for j in range(C // GS):
            sl = pl.ds(j * GS, GS)
            descs.append(
                pltpu.async_copy(y_hbm.at[srcb.at[sl]], ysrc.at[sl], sem))
            descs.append(
                pltpu.async_copy(y_hbm.at[dstb.at[sl]], ydst.at[sl], sem))
        for dsc in descs:
            dsc.wait()

        @plsc.parallel_loop(0, C // 16, unroll=2)
        def _wgt(g):
            gb = g * 16
            dv = dstb[pl.ds(gb, 16)] - base
            ok = (dv >= 0) & (dv < NT)
            rowb[pl.ds(gb, 16)] = jnp.where(ok, dv, NT)
            ridx = gb + iota16
            a0 = jnp.zeros((16,), jnp.float32)
            a1 = jnp.zeros((16,), jnp.float32)
            a2 = jnp.zeros((16,), jnp.float32)
            a3 = jnp.zeros((16,), jnp.float32)
            for d in range(0, DP, 4):
                d0 = plsc.load_gather(ysrc, [ridx, _splat16(d)]) - \
                    plsc.load_gather(ydst, [ridx, _splat16(d)])
                d1 = plsc.load_gather(ysrc, [ridx, _splat16(d + 1)]) - \
                    plsc.load_gather(ydst, [ridx, _splat16(d + 1)])
                d2 = plsc.load_gather(ysrc, [ridx, _splat16(d + 2)]) - \
                    plsc.load_gather(ydst, [ridx, _splat16(d + 2)])
                d3 = plsc.load_gather(ysrc, [ridx, _splat16(d + 3)]) - \
                    plsc.load_gather(ydst, [ridx, _splat16(d + 3)])
                a0 = a0 + d0 * d0
                a1 = a1 + d1 * d1
                a2 = a2 + d2 * d2
                a3 = a3 + d3 * d3
            x = (a0 + a1) + (a2 + a3) + 1.0
            ib = plsc.bitcast(x, jnp.int32)
            yv = plsc.bitcast(jnp.int32(0x5F3759DF) - (ib >> 1), jnp.float32)
            for _ in range(3):
                yv = yv * (1.5 - 0.5 * x * yv * yv)
            wsb[pl.ds(gb, 16)] = yv * wnb[pl.ds(gb, 16)]

        @plsc.parallel_loop(0, C, unroll=4)
        def _scat(e):
            rsp = plsc.load_gather(rowb, [_splat16(e)])
            wsp = plsc.load_gather(wsb, [_splat16(e)])
            lo = ysrc[e, pl.ds(0, 16)]
            hi = ysrc[e, pl.ds(16, 16)]
            plsc.addupdate_scatter(agg, [rsp, iota16], lo * wsp)
            plsc.addupdate_scatter(agg, [rsp, iota16 + 16], hi * wsp)
        return 0

    lax.fori_loop(0, nch, _chunk, 0)

    def _upd(b, _):
        rb = base + b * BU
        pltpu.sync_copy(h_hbm.at[pl.ds(rb, BU)], hbuf)

        @plsc.parallel_loop(0, BU, unroll=4)
        def _row(r):
            ar = b * BU + r
            ybuf[r, pl.ds(0, 16)] = (hbuf[r, pl.ds(0, 16)]
                                     + agg[ar, pl.ds(0, 16)]) * 0.5
            ybuf[r, pl.ds(16, 16)] = (hbuf[r, pl.ds(16, 16)]
                                      + agg[ar, pl.ds(16, 16)]) * 0.5
        pltpu.sync_copy(ybuf, ynew_hbm.at[pl.ds(rb, BU)])
        return 0
    lax.fori_loop(0, NT // BU, _upd, 0)


@functools.partial(
    pl.kernel,
    out_type=jax.ShapeDtypeStruct((N_PAD, DP), jnp.float32),
    mesh=plsc.VectorSubcoreMesh(core_axis_name="c", subcore_axis_name="s"),
    scratch_types=[
        pltpu.VMEM((16,), jnp.int32),
        pltpu.VMEM((C,), jnp.int32),
        pltpu.VMEM((C,), jnp.int32),
        pltpu.VMEM((C,), jnp.float32),
        pltpu.VMEM((C,), jnp.int32),
        pltpu.VMEM((C,), jnp.float32),
        pltpu.VMEM((C, DP), jnp.float32),
        pltpu.VMEM((C, DP), jnp.float32),
        pltpu.VMEM((NT + 8, DP), jnp.float32),
        pltpu.VMEM((BU, DP), jnp.float32),
        pltpu.VMEM((BU, DP), jnp.float32),
        pltpu.SemaphoreType.DMA,
    ],
    compiler_params=pltpu.CompilerParams(needs_layout_passes=False,
                                         use_tc_tiling_on_sc=False),
)
def _prop_step(y_hbm, h_hbm, src_hbm, dst_hbm, wn_hbm, starts_hbm,
               ynew_hbm, *scratch):
    _step_body(y_hbm, h_hbm, src_hbm, dst_hbm, wn_hbm, starts_hbm,
               ynew_hbm, *scratch)


def _lstm_dir(xs, Wih, Whh, bih, bhh, reverse):
    def step(carry, xt):
        h, c = carry
        g = xt @ Wih.T + h @ Whh.T + bih + bhh
        i, f, gg, o = jnp.split(g, 4)
        i = jax.nn.sigmoid(i)
        f = jax.nn.sigmoid(f)
        gg = jnp.tanh(gg)
        o = jax.nn.sigmoid(o)
        c = f * c + i * gg
        h = o * jnp.tanh(c)
        return (h, c), h
    seq = xs[::-1] if reverse else xs
    init = (jnp.zeros((D,), xs.dtype), jnp.zeros((D,), xs.dtype))
    _, hs = jax.lax.scan(step, init, seq)
    return hs[::-1] if reverse else hs


def kernel(x, edge_index, ligand_emb, params):
    src = edge_index[0]
    dst = edge_index[1]

    order = jnp.argsort(dst)
    dst_s = dst[order]
    src_s = src[order]
    grid = jnp.arange(N + 1, dtype=jnp.int32)
    b_in = jnp.searchsorted(dst_s, grid)
    deg_in = jnp.diff(b_in)
    src_sorted = jnp.sort(src)
    b_out = jnp.searchsorted(src_sorted, grid)
    deg_out = jnp.diff(b_out)
    norm_s = jax.lax.rsqrt(
        (deg_out[src_s].astype(jnp.float32) + 1.0)
        * (deg_in[dst_s].astype(jnp.float32) + 1.0))

    tbound = jnp.searchsorted(dst_s, jnp.arange(33, dtype=jnp.int32) * NT)
    starts = jnp.full((40,), E, jnp.int32).at[:33].set(tbound.astype(jnp.int32))
    starts_b = jnp.broadcast_to(starts[:, None], (40, 16)).astype(jnp.int32)

    pad_e = E_PAD - E
    src_p = jnp.concatenate([src_s.astype(jnp.int32),
                             jnp.zeros((pad_e,), jnp.int32)])
    dst_p = jnp.concatenate([dst_s.astype(jnp.int32),
                             jnp.full((pad_e,), N_PAD - 1, jnp.int32)])
    wn_p = jnp.concatenate([norm_s, jnp.zeros((pad_e,), jnp.float32)])

    def pad_feat(f31):
        return jnp.pad(f31, ((0, N_PAD - N), (0, DP - D)))

    feat31 = x
    for i in range(NLAYERS):
        h31 = jax.nn.relu(feat31 @ params['convW1'][i] + params['convb1'][i])
        h31 = h31 @ params['convW2'][i] + params['convb2'][i]
        hp = pad_feat(h31)
        Y = hp
        for _ in range(7):
            Y = _prop_step(Y, hp, src_p, dst_p, wn_p, starts_b)
        feat31 = jax.nn.relu(Y[:N, :D])

    gate = feat31 @ params['pool_w'] + params['pool_b']
    a = jax.nn.softmax(gate, axis=0)
    protein_rep = jnp.sum(a * feat31, axis=0, keepdims=True)

    seq = jnp.concatenate([ligand_emb, protein_rep], axis=0)
    mask = jnp.eye(L, dtype=jnp.float32)
    mask = mask.at[:, L - 1].set(1.0)
    mask = mask.at[L - 1, :].set(1.0)
    mask = mask.at[L - 1, L - 1].set(0.0)
    h = seq
    for layer in (0, 1):
        sf = '_l' + str(layer) + 'f'
        sb = '_l' + str(layer) + 'b'
        hf = _lstm_dir(h, params['Wih' + sf], params['Whh' + sf],
                       params['bih' + sf], params['bhh' + sf], False)
        hb = _lstm_dir(h, params['Wih' + sb], params['Whh' + sb],
                       params['bih' + sb], params['bhh' + sb], True)
        h = jnp.concatenate([hf, hb], axis=-1)
    out = h[None]
    dh = DH // NH
    q = (out @ params['Wq'] + params['bq']).reshape(1, L, NH, dh).transpose(0, 2, 1, 3)
    k = (out @ params['Wk'] + params['bk']).reshape(1, L, NH, dh).transpose(0, 2, 1, 3)
    v = (out @ params['Wv'] + params['bv']).reshape(1, L, NH, dh).transpose(0, 2, 1, 3)
    scores = (q @ k.transpose(0, 1, 3, 2)) / float(np.sqrt(dh))
    scores = jnp.where(mask[None, None, :, :] > 0, scores, -1e9)
    attn = jax.nn.softmax(scores, axis=-1)
    ctx = (attn @ v).transpose(0, 2, 1, 3).reshape(1, L, DH)
    ctx = ctx @ params['Wo'] + params['bo']
    flat = ctx.reshape(1, L * DH)
    hid = jax.nn.relu(flat @ params['fc_in_w'] + params['fc_in_b'])
    return jax.nn.sigmoid(hid @ params['fc_out_w'] + params['fc_out_b'])


# speedup vs baseline: 4.0577x; 1.1484x over previous
import functools

import jax
import jax.numpy as jnp
import numpy as np
from jax import lax
from jax.experimental import pallas as pl
from jax.experimental.pallas import tpu as pltpu
from jax.experimental.pallas import tpu_sc as plsc

N = 50000
E = 800000
D = 31
HID = 8
NLAYERS = 14
LIG = 139
L = 140
DH = 62
NH = 2

NTILES = 32
NT = 1568
N_PAD = NTILES * NT
DP = 32
C = 640
GS = 128
E_PAD = E + 1024
BU = 224


def _splat16(s):
    return jnp.full((16,), s, jnp.int32)


def _step_body(y_hbm, h_hbm, src_hbm, dst_hbm, wn_hbm, starts_hbm,
               ynew_hbm,
               sbuf, srcb, dstb, wnb, rowb, wsb, ysrc, ylocal, agg, sem):
    wid = lax.axis_index("s") * 2 + lax.axis_index("c")
    base = wid * NT

    zero16 = jnp.zeros((16,), jnp.float32)

    @plsc.parallel_loop(0, NT + 8, unroll=4)
    def _zero(r):
        agg[r, pl.ds(0, 16)] = zero16
        agg[r, pl.ds(16, 16)] = zero16

    pltpu.sync_copy(y_hbm.at[pl.ds(base, NT)], ylocal.at[pl.ds(0, NT)])

    pltpu.sync_copy(starts_hbm.at[wid], sbuf)
    start = sbuf[...][0]
    pltpu.sync_copy(starts_hbm.at[wid + 1], sbuf)
    end = sbuf[...][0]
    start_al = (start // 8) * 8
    nch = (end - start_al + (C - 1)) // C

    iota16 = lax.iota(jnp.int32, 16)

    def _chunk(c, _):
        off = start_al + c * C
        e1 = pltpu.async_copy(src_hbm.at[pl.ds(off, C)], srcb, sem)
        e2 = pltpu.async_copy(dst_hbm.at[pl.ds(off, C)], dstb, sem)
        e3 = pltpu.async_copy(wn_hbm.at[pl.ds(off, C)], wnb, sem)
        e1.wait()
        e2.wait()
        e3.wait()
        descs = []
        for j in range(C // GS):
            sl = pl.ds(j * GS, GS)
            descs.append(
                pltpu.async_copy(y_hbm.at[srcb.at[sl]], ysrc.at[sl], sem))
        for dsc in descs:
            dsc.wait()

        @plsc.parallel_loop(0, C // 16, unroll=2)
        def _wgt(g):
            gb = g * 16
            dv = dstb[pl.ds(gb, 16)] - base
            ok = (dv >= 0) & (dv < NT)
            rloc = jnp.where(ok, dv, NT)
            rowb[pl.ds(gb, 16)] = rloc
            ridx = gb + iota16
            a0 = jnp.zeros((16,), jnp.float32)
            a1 = jnp.zeros((16,), jnp.float32)
            a2 = jnp.zeros((16,), jnp.float32)
            a3 = jnp.zeros((16,), jnp.float32)
            for d in range(0, DP, 4):
                d0 = plsc.load_gather(ysrc, [ridx, _splat16(d)]) - \
                    plsc.load_gather(ylocal, [rloc, _splat16(d)])
                d1 = plsc.load_gather(ysrc, [ridx, _splat16(d + 1)]) - \
                    plsc.load_gather(ylocal, [rloc, _splat16(d + 1)])
                d2 = plsc.load_gather(ysrc, [ridx, _splat16(d + 2)]) - \
                    plsc.load_gather(ylocal, [rloc, _splat16(d + 2)])
                d3 = plsc.load_gather(ysrc, [ridx, _splat16(d + 3)]) - \
                    plsc.load_gather(ylocal, [rloc, _splat16(d + 3)])
                a0 = a0 + d0 * d0
                a1 = a1 + d1 * d1
                a2 = a2 + d2 * d2
                a3 = a3 + d3 * d3
            x = (a0 + a1) + (a2 + a3) + 1.0
            ib = plsc.bitcast(x, jnp.int32)
            yv = plsc.bitcast(jnp.int32(0x5F3759DF) - (ib >> 1), jnp.float32)
            for _ in range(3):
                yv = yv * (1.5 - 0.5 * x * yv * yv)
            wsb[pl.ds(gb, 16)] = yv * wnb[pl.ds(gb, 16)]

        @plsc.parallel_loop(0, C, unroll=4)
        def _scat(e):
            rsp = plsc.load_gather(rowb, [_splat16(e)])
            wsp = plsc.load_gather(wsb, [_splat16(e)])
            lo = ysrc[e, pl.ds(0, 16)]
            hi = ysrc[e, pl.ds(16, 16)]
            plsc.addupdate_scatter(agg, [rsp, iota16], lo * wsp)
            plsc.addupdate_scatter(agg, [rsp, iota16 + 16], hi * wsp)
        return 0

    lax.fori_loop(0, nch, _chunk, 0)

    def _upd(b, _):
        rb = base + b * BU
        pltpu.sync_copy(h_hbm.at[pl.ds(rb, BU)], ysrc.at[pl.ds(0, BU)])

        @plsc.parallel_loop(0, BU, unroll=4)
        def _row(r):
            ar = b * BU + r
            ysrc[r, pl.ds(0, 16)] = (ysrc[r, pl.ds(0, 16)]
                                     + agg[ar, pl.ds(0, 16)]) * 0.5
            ysrc[r, pl.ds(16, 16)] = (ysrc[r, pl.ds(16, 16)]
                                      + agg[ar, pl.ds(16, 16)]) * 0.5
        pltpu.sync_copy(ysrc.at[pl.ds(0, BU)], ynew_hbm.at[pl.ds(rb, BU)])
        return 0
    lax.fori_loop(0, NT // BU, _upd, 0)


@functools.partial(
    pl.kernel,
    out_type=jax.ShapeDtypeStruct((N_PAD, DP), jnp.float32),
    mesh=plsc.VectorSubcoreMesh(core_axis_name="c", subcore_axis_name="s"),
    scratch_types=[
        pltpu.VMEM((16,), jnp.int32),
        pltpu.VMEM((C,), jnp.int32),
        pltpu.VMEM((C,), jnp.int32),
        pltpu.VMEM((C,), jnp.float32),
        pltpu.VMEM((C,), jnp.int32),
        pltpu.VMEM((C,), jnp.float32),
        pltpu.VMEM((C, DP), jnp.float32),
        pltpu.VMEM((NT + 8, DP), jnp.float32),
        pltpu.VMEM((NT + 8, DP), jnp.float32),
        pltpu.SemaphoreType.DMA,
    ],
    compiler_params=pltpu.CompilerParams(needs_layout_passes=False,
                                         use_tc_tiling_on_sc=False),
)
def _prop_step(y_hbm, h_hbm, src_hbm, dst_hbm, wn_hbm, starts_hbm,
               ynew_hbm, *scratch):
    _step_body(y_hbm, h_hbm, src_hbm, dst_hbm, wn_hbm, starts_hbm,
               ynew_hbm, *scratch)


def _lstm_dir(xs, Wih, Whh, bih, bhh, reverse):
    def step(carry, xt):
        h, c = carry
        g = xt @ Wih.T + h @ Whh.T + bih + bhh
        i, f, gg, o = jnp.split(g, 4)
        i = jax.nn.sigmoid(i)
        f = jax.nn.sigmoid(f)
        gg = jnp.tanh(gg)
        o = jax.nn.sigmoid(o)
        c = f * c + i * gg
        h = o * jnp.tanh(c)
        return (h, c), h
    seq = xs[::-1] if reverse else xs
    init = (jnp.zeros((D,), xs.dtype), jnp.zeros((D,), xs.dtype))
    _, hs = jax.lax.scan(step, init, seq)
    return hs[::-1] if reverse else hs


def kernel(x, edge_index, ligand_emb, params):
    src = edge_index[0]
    dst = edge_index[1]

    order = jnp.argsort(dst)
    dst_s = dst[order]
    src_s = src[order]
    grid = jnp.arange(N + 1, dtype=jnp.int32)
    b_in = jnp.searchsorted(dst_s, grid)
    deg_in = jnp.diff(b_in)
    src_sorted = jnp.sort(src)
    b_out = jnp.searchsorted(src_sorted, grid)
    deg_out = jnp.diff(b_out)
    norm_s = jax.lax.rsqrt(
        (deg_out[src_s].astype(jnp.float32) + 1.0)
        * (deg_in[dst_s].astype(jnp.float32) + 1.0))

    tbound = jnp.searchsorted(dst_s, jnp.arange(33, dtype=jnp.int32) * NT)
    starts = jnp.full((40,), E, jnp.int32).at[:33].set(tbound.astype(jnp.int32))
    starts_b = jnp.broadcast_to(starts[:, None], (40, 16)).astype(jnp.int32)

    pad_e = E_PAD - E
    src_p = jnp.concatenate([src_s.astype(jnp.int32),
                             jnp.zeros((pad_e,), jnp.int32)])
    dst_p = jnp.concatenate([dst_s.astype(jnp.int32),
                             jnp.full((pad_e,), N_PAD - 1, jnp.int32)])
    wn_p = jnp.concatenate([norm_s, jnp.zeros((pad_e,), jnp.float32)])

    def pad_feat(f31):
        return jnp.pad(f31, ((0, N_PAD - N), (0, DP - D)))

    feat31 = x
    for i in range(NLAYERS):
        h31 = jax.nn.relu(feat31 @ params['convW1'][i] + params['convb1'][i])
        h31 = h31 @ params['convW2'][i] + params['convb2'][i]
        hp = pad_feat(h31)
        Y = hp
        for _ in range(7):
            Y = _prop_step(Y, hp, src_p, dst_p, wn_p, starts_b)
        feat31 = jax.nn.relu(Y[:N, :D])

    gate = feat31 @ params['pool_w'] + params['pool_b']
    a = jax.nn.softmax(gate, axis=0)
    protein_rep = jnp.sum(a * feat31, axis=0, keepdims=True)

    seq = jnp.concatenate([ligand_emb, protein_rep], axis=0)
    mask = jnp.eye(L, dtype=jnp.float32)
    mask = mask.at[:, L - 1].set(1.0)
    mask = mask.at[L - 1, :].set(1.0)
    mask = mask.at[L - 1, L - 1].set(0.0)
    h = seq
    for layer in (0, 1):
        sf = '_l' + str(layer) + 'f'
        sb = '_l' + str(layer) + 'b'
        hf = _lstm_dir(h, params['Wih' + sf], params['Whh' + sf],
                       params['bih' + sf], params['bhh' + sf], False)
        hb = _lstm_dir(h, params['Wih' + sb], params['Whh' + sb],
                       params['bih' + sb], params['bhh' + sb], True)
        h = jnp.concatenate([hf, hb], axis=-1)
    out = h[None]
    dh = DH // NH
    q = (out @ params['Wq'] + params['bq']).reshape(1, L, NH, dh).transpose(0, 2, 1, 3)
    k = (out @ params['Wk'] + params['bk']).reshape(1, L, NH, dh).transpose(0, 2, 1, 3)
    v = (out @ params['Wv'] + params['bv']).reshape(1, L, NH, dh).transpose(0, 2, 1, 3)
    scores = (q @ k.transpose(0, 1, 3, 2)) / float(np.sqrt(dh))
    scores = jnp.where(mask[None, None, :, :] > 0, scores, -1e9)
    attn = jax.nn.softmax(scores, axis=-1)
    ctx = (attn @ v).transpose(0, 2, 1, 3).reshape(1, L, DH)
    ctx = ctx @ params['Wo'] + params['bo']
    flat = ctx.reshape(1, L * DH)
    hid = jax.nn.relu(flat @ params['fc_in_w'] + params['fc_in_b'])
    return jax.nn.sigmoid(hid @ params['fc_out_w'] + params['fc_out_b'])


# speedup vs baseline: 7.0353x; 1.7338x over previous
import functools

import jax
import jax.numpy as jnp
import numpy as np
from jax import lax
from jax.experimental import pallas as pl
from jax.experimental.pallas import tpu as pltpu
from jax.experimental.pallas import tpu_sc as plsc

N = 50000
E = 800000
D = 31
HID = 8
NLAYERS = 14
LIG = 139
L = 140
DH = 62
NH = 2

NTILES = 32
NT = 1568
N_PAD = NTILES * NT
DP = 32
C = 640
GS = 128
E_PAD = E + 1024
BU = 224


def _splat16(s):
    return jnp.full((16,), s, jnp.int32)


def _step_body(y_hbm, h_hbm, src_hbm, dst_hbm, wn_hbm, starts_hbm,
               ynew_hbm,
               sbuf, srcb, dstb, wnb, ysrc, ylocal, agg, sem):
    wid = lax.axis_index("s") * 2 + lax.axis_index("c")
    base = wid * NT

    zero16 = jnp.zeros((16,), jnp.float32)

    @plsc.parallel_loop(0, NT + 8, unroll=4)
    def _zero(r):
        agg[r, pl.ds(0, 16)] = zero16
        agg[r, pl.ds(16, 16)] = zero16

    pltpu.sync_copy(y_hbm.at[pl.ds(base, NT)], ylocal.at[pl.ds(0, NT)])

    pltpu.sync_copy(starts_hbm.at[wid], sbuf)
    start = sbuf[...][0]
    pltpu.sync_copy(starts_hbm.at[wid + 1], sbuf)
    end = sbuf[...][0]
    start_al = (start // 8) * 8
    nch = (end - start_al + (C - 1)) // C

    iota16 = lax.iota(jnp.int32, 16)

    def _chunk(c, _):
        off = start_al + c * C
        e1 = pltpu.async_copy(src_hbm.at[pl.ds(off, C)], srcb, sem)
        e2 = pltpu.async_copy(dst_hbm.at[pl.ds(off, C)], dstb, sem)
        e3 = pltpu.async_copy(wn_hbm.at[pl.ds(off, C)], wnb, sem)
        e1.wait()
        e2.wait()
        e3.wait()
        descs = []
        for j in range(C // GS):
            sl = pl.ds(j * GS, GS)
            descs.append(
                pltpu.async_copy(y_hbm.at[srcb.at[sl]], ysrc.at[sl], sem))
        for dsc in descs:
            dsc.wait()

        p8 = iota16 ^ 8
        p4 = iota16 ^ 4
        p2 = iota16 ^ 2
        p1 = iota16 ^ 1

        @plsc.parallel_loop(0, C // 16, unroll=1)
        def _grp(g):
            gb = g * 16
            dstv = dstb[pl.ds(gb, 16)]
            wnv = wnb[pl.ds(gb, 16)]
            dv = dstv - base
            ok = (dv >= 0) & (dv < NT)
            rlocv = jnp.where(ok, dv, NT)
            x = jnp.zeros((16,), jnp.float32)
            for k in range(16):
                e = gb + k
                r = rlocv[k]
                d0 = ysrc[e, pl.ds(0, 16)] - ylocal[r, pl.ds(0, 16)]
                d1 = ysrc[e, pl.ds(16, 16)] - ylocal[r, pl.ds(16, 16)]
                s = d0 * d0 + d1 * d1
                s = s + s[p8]
                s = s + s[p4]
                s = s + s[p2]
                s = s + s[p1]
                x = jnp.where(iota16 == k, s, x)
            x = x + 1.0
            ib = plsc.bitcast(x, jnp.int32)
            yv = plsc.bitcast(jnp.int32(0x5F3759DF) - (ib >> 1), jnp.float32)
            for _ in range(3):
                yv = yv * (1.5 - 0.5 * x * yv * yv)
            wv = yv * wnv
            for k in range(16):
                e = gb + k
                rs = jnp.full((16,), rlocv[k], jnp.int32)
                ws = jnp.full((16,), wv[k], jnp.float32)
                lo = ysrc[e, pl.ds(0, 16)]
                hi = ysrc[e, pl.ds(16, 16)]
                plsc.addupdate_scatter(agg, [rs, iota16], lo * ws)
                plsc.addupdate_scatter(agg, [rs, iota16 + 16], hi * ws)
        return 0

    lax.fori_loop(0, nch, _chunk, 0)

    def _upd(b, _):
        rb = base + b * BU
        pltpu.sync_copy(h_hbm.at[pl.ds(rb, BU)], ysrc.at[pl.ds(0, BU)])

        @plsc.parallel_loop(0, BU, unroll=4)
        def _row(r):
            ar = b * BU + r
            ysrc[r, pl.ds(0, 16)] = (ysrc[r, pl.ds(0, 16)]
                                     + agg[ar, pl.ds(0, 16)]) * 0.5
            ysrc[r, pl.ds(16, 16)] = (ysrc[r, pl.ds(16, 16)]
                                      + agg[ar, pl.ds(16, 16)]) * 0.5
        pltpu.sync_copy(ysrc.at[pl.ds(0, BU)], ynew_hbm.at[pl.ds(rb, BU)])
        return 0
    lax.fori_loop(0, NT // BU, _upd, 0)


@functools.partial(
    pl.kernel,
    out_type=jax.ShapeDtypeStruct((N_PAD, DP), jnp.float32),
    mesh=plsc.VectorSubcoreMesh(core_axis_name="c", subcore_axis_name="s"),
    scratch_types=[
        pltpu.VMEM((16,), jnp.int32),
        pltpu.VMEM((C,), jnp.int32),
        pltpu.VMEM((C,), jnp.int32),
        pltpu.VMEM((C,), jnp.float32),
        pltpu.VMEM((C, DP), jnp.float32),
        pltpu.VMEM((NT + 8, DP), jnp.float32),
        pltpu.VMEM((NT + 8, DP), jnp.float32),
        pltpu.SemaphoreType.DMA,
    ],
    compiler_params=pltpu.CompilerParams(needs_layout_passes=False,
                                         use_tc_tiling_on_sc=False),
)
def _prop_step(y_hbm, h_hbm, src_hbm, dst_hbm, wn_hbm, starts_hbm,
               ynew_hbm, *scratch):
    _step_body(y_hbm, h_hbm, src_hbm, dst_hbm, wn_hbm, starts_hbm,
               ynew_hbm, *scratch)


def _lstm_dir(xs, Wih, Whh, bih, bhh, reverse):
    def step(carry, xt):
        h, c = carry
        g = xt @ Wih.T + h @ Whh.T + bih + bhh
        i, f, gg, o = jnp.split(g, 4)
        i = jax.nn.sigmoid(i)
        f = jax.nn.sigmoid(f)
        gg = jnp.tanh(gg)
        o = jax.nn.sigmoid(o)
        c = f * c + i * gg
        h = o * jnp.tanh(c)
        return (h, c), h
    seq = xs[::-1] if reverse else xs
    init = (jnp.zeros((D,), xs.dtype), jnp.zeros((D,), xs.dtype))
    _, hs = jax.lax.scan(step, init, seq)
    return hs[::-1] if reverse else hs


def kernel(x, edge_index, ligand_emb, params):
    src = edge_index[0]
    dst = edge_index[1]

    order = jnp.argsort(dst)
    dst_s = dst[order]
    src_s = src[order]
    grid = jnp.arange(N + 1, dtype=jnp.int32)
    b_in = jnp.searchsorted(dst_s, grid)
    deg_in = jnp.diff(b_in)
    src_sorted = jnp.sort(src)
    b_out = jnp.searchsorted(src_sorted, grid)
    deg_out = jnp.diff(b_out)
    norm_s = jax.lax.rsqrt(
        (deg_out[src_s].astype(jnp.float32) + 1.0)
        * (deg_in[dst_s].astype(jnp.float32) + 1.0))

    tbound = jnp.searchsorted(dst_s, jnp.arange(33, dtype=jnp.int32) * NT)
    starts = jnp.full((40,), E, jnp.int32).at[:33].set(tbound.astype(jnp.int32))
    starts_b = jnp.broadcast_to(starts[:, None], (40, 16)).astype(jnp.int32)

    pad_e = E_PAD - E
    src_p = jnp.concatenate([src_s.astype(jnp.int32),
                             jnp.zeros((pad_e,), jnp.int32)])
    dst_p = jnp.concatenate([dst_s.astype(jnp.int32),
                             jnp.full((pad_e,), N_PAD - 1, jnp.int32)])
    wn_p = jnp.concatenate([norm_s, jnp.zeros((pad_e,), jnp.float32)])

    def pad_feat(f31):
        return jnp.pad(f31, ((0, N_PAD - N), (0, DP - D)))

    feat31 = x
    for i in range(NLAYERS):
        h31 = jax.nn.relu(feat31 @ params['convW1'][i] + params['convb1'][i])
        h31 = h31 @ params['convW2'][i] + params['convb2'][i]
        hp = pad_feat(h31)
        Y = hp
        for _ in range(7):
            Y = _prop_step(Y, hp, src_p, dst_p, wn_p, starts_b)
        feat31 = jax.nn.relu(Y[:N, :D])

    gate = feat31 @ params['pool_w'] + params['pool_b']
    a = jax.nn.softmax(gate, axis=0)
    protein_rep = jnp.sum(a * feat31, axis=0, keepdims=True)

    seq = jnp.concatenate([ligand_emb, protein_rep], axis=0)
    mask = jnp.eye(L, dtype=jnp.float32)
    mask = mask.at[:, L - 1].set(1.0)
    mask = mask.at[L - 1, :].set(1.0)
    mask = mask.at[L - 1, L - 1].set(0.0)
    h = seq
    for layer in (0, 1):
        sf = '_l' + str(layer) + 'f'
        sb = '_l' + str(layer) + 'b'
        hf = _lstm_dir(h, params['Wih' + sf], params['Whh' + sf],
                       params['bih' + sf], params['bhh' + sf], False)
        hb = _lstm_dir(h, params['Wih' + sb], params['Whh' + sb],
                       params['bih' + sb], params['bhh' + sb], True)
        h = jnp.concatenate([hf, hb], axis=-1)
    out = h[None]
    dh = DH // NH
    q = (out @ params['Wq'] + params['bq']).reshape(1, L, NH, dh).transpose(0, 2, 1, 3)
    k = (out @ params['Wk'] + params['bk']).reshape(1, L, NH, dh).transpose(0, 2, 1, 3)
    v = (out @ params['Wv'] + params['bv']).reshape(1, L, NH, dh).transpose(0, 2, 1, 3)
    scores = (q @ k.transpose(0, 1, 3, 2)) / float(np.sqrt(dh))
    scores = jnp.where(mask[None, None, :, :] > 0, scores, -1e9)
    attn = jax.nn.softmax(scores, axis=-1)
    ctx = (attn @ v).transpose(0, 2, 1, 3).reshape(1, L, DH)
    ctx = ctx @ params['Wo'] + params['bo']
    flat = ctx.reshape(1, L * DH)
    hid = jax.nn.relu(flat @ params['fc_in_w'] + params['fc_in_b'])
    return jax.nn.sigmoid(hid @ params['fc_out_w'] + params['fc_out_b'])
